# baseline (speedup 1.0000x reference)
import functools
import jax
import jax.numpy as jnp
from jax import lax
from jax.experimental import pallas as pl
from jax.experimental.pallas import tpu as pltpu
from jax.experimental.pallas import tpu_sc as plsc

_NUM_MOD = 3
_BLOCK = 8192


def _experts_kernel(x_ref, m_ref, w_ref, y_ref, idx_ref):
    i = pl.program_id(0)
    xb = x_ref[...]
    m = m_ref[0, 0, :]
    w = w_ref[...]
    n_total = pl.num_programs(0) * xb.shape[0]
    for e in range(_NUM_MOD):
        ye = lax.dot_general(
            xb, w[e], (((1,), (1,)), ((), ())),
            preferred_element_type=jnp.float32)
        y_ref[e, :, :] = jnp.concatenate(
            [ye, jnp.zeros_like(ye)], axis=1)
    base = i * xb.shape[0]
    tok = base + lax.broadcasted_iota(jnp.int32, (1, 1, xb.shape[0]), 2)
    idx_ref[...] = m[None, None, :] * n_total + tok


def _make_sc_gather(v, d, b):
    info = plsc.get_sparse_core_info()
    nw = info.num_cores * info.num_subcores
    assert d % info.num_lanes == 0 and b % (8 * nw) == 0
    b_per_w = b // nw
    mesh = plsc.VectorSubcoreMesh(core_axis_name="c", subcore_axis_name="s")

    @functools.partial(
        pl.kernel, mesh=mesh,
        out_type=jax.ShapeDtypeStruct((b, 2 * d), jnp.float32),
        scratch_types=[
            pltpu.VMEM((b_per_w,), jnp.int32),
            pltpu.VMEM((b_per_w, 2 * d), jnp.float32),
            pltpu.SemaphoreType.DMA,
        ],
    )
    def gather(table_hbm, idx_hbm, out_hbm, idx_v, rows_v, sem):
        wid = lax.axis_index("s") * info.num_cores + lax.axis_index("c")
        base = wid * b_per_w
        pltpu.sync_copy(idx_hbm.at[pl.ds(base, b_per_w)], idx_v)
        pltpu.async_copy(table_hbm.at[idx_v], rows_v, sem).wait()
        pltpu.sync_copy(rows_v, out_hbm.at[pl.ds(base, b_per_w)])

    return gather


def kernel(x, modality_mapping, W):
    n, h = x.shape
    b = _BLOCK
    nblk = n // b
    m3 = modality_mapping.reshape(nblk, 1, b)
    ycat, idx3 = pl.pallas_call(
        _experts_kernel,
        grid=(nblk,),
        in_specs=[
            pl.BlockSpec((b, h), lambda i: (i, 0)),
            pl.BlockSpec((1, 1, b), lambda i: (i, 0, 0)),
            pl.BlockSpec((_NUM_MOD, h, h), lambda i: (0, 0, 0)),
        ],
        out_specs=[
            pl.BlockSpec((_NUM_MOD, b, 2 * h), lambda i: (0, i, 0)),
            pl.BlockSpec((1, 1, b), lambda i: (i, 0, 0)),
        ],
        out_shape=[
            jax.ShapeDtypeStruct((_NUM_MOD, n, 2 * h), jnp.float32),
            jax.ShapeDtypeStruct((nblk, 1, b), jnp.int32),
        ],
    )(x, m3, W)
    table = ycat.reshape(_NUM_MOD * n, 2 * h)
    idx = idx3.reshape(n)
    wide = _make_sc_gather(_NUM_MOD * n, h, n)(table, idx)
    return wide[:, :h]
